# baseline (speedup 1.0000x reference)
import functools

import jax
import jax.numpy as jnp
from jax import lax
from jax.experimental import pallas as pl
from jax.experimental.pallas import tpu as pltpu
from jax.experimental.pallas import tpu_sc as plsc

N_NODES = 10000
N_PAD = 10240
D_FEAT = 128
HF = 64
E_EDGES = 320000
E_PAD = 331776
C = 128
CH = E_PAD // (16 * C)
NIB = 6
RB = 5
DT_MIN = 0.01
DT_MAX = 2.0

_f32 = jnp.float32
_i32 = jnp.int32
_bf16 = jnp.bfloat16

_RING = []
for _ in range(NIB):
    _RING += [
        pltpu.VMEM((1, C), _i32),
        pltpu.VMEM((1, C), _i32),
        pltpu.VMEM((C,), _f32),
        pltpu.VMEM((C,), _f32),
    ]


@functools.partial(
    pl.kernel,
    out_type=[jax.ShapeDtypeStruct((N_PAD, HF), _f32) for _ in range(2)],
    mesh=plsc.VectorSubcoreMesh(core_axis_name="c", subcore_axis_name="s"),
    compiler_params=pltpu.CompilerParams(use_tc_tiling_on_sc=False,
                                         needs_layout_passes=False),
    scratch_types=[
        pltpu.VMEM((C, HF), _bf16),
        pltpu.VMEM((C, HF), _bf16),
        pltpu.VMEM((C, HF), _bf16),
        pltpu.VMEM((C, HF), _bf16),
        pltpu.VMEM((C, HF), _f32),
        pltpu.VMEM((C, HF), _f32),
        pltpu.VMEM((C, HF), _f32),
        pltpu.VMEM((C,), _f32),
        pltpu.VMEM((C,), _f32),
        pltpu.VMEM((16,), _f32),
        pltpu.VMEM((16,), _f32),
        pltpu.VMEM((C,), _f32),
        pltpu.VMEM_SHARED((N_PAD, HF), _f32),
        pltpu.VMEM_SHARED((N_PAD,), _f32),
        pltpu.SemaphoreType.DMA,
        pltpu.SemaphoreType.DMA,
        pltpu.SemaphoreType.DMA,
        pltpu.SemaphoreType.DMA,
        pltpu.SemaphoreType.DMA,
        pltpu.SemaphoreType.DMA,
        pltpu.SemaphoreType.DMA,
        pltpu.SemaphoreType.DMA,
    ] + _RING,
)
def _gstep_sc(ub0, ub1, uh0, uh1, dstr, srcr, dxr, dzr, dt_h, g_h,
              oh0, oh1,
              udA, usA, udB, usB, difA, difB, ndif, slpcA, slpcB,
              dtb, gb, sbuf, acc, sacc,
              semF, semGA, semGB, semDA, semDB, semN, semSA, semSB, *ring):
    idxD = [ring[4 * n + 0] for n in range(NIB)]
    idxS = [ring[4 * n + 1] for n in range(NIB)]
    dxc = [ring[4 * n + 2] for n in range(NIB)]
    dzc = [ring[4 * n + 3] for n in range(NIB)]
    ud = [udA, udB]
    us = [usA, usB]
    dif = [difA, difB]
    slpc = [slpcA, slpcB]
    semG = [semGA, semGB]
    semD = [semDA, semDB]
    semS = [semSA, semSB]

    c = lax.axis_index("c")
    s = lax.axis_index("s")

    pltpu.sync_copy(dt_h, dtb)
    pltpu.sync_copy(g_h, gb)
    dtc = jnp.minimum(jnp.maximum(dtb[...], DT_MIN), DT_MAX)
    gdt = gb[...] * dtc

    base = s * (RB * C)
    eoff = s * CH

    def _edge_phase(ub_hbm):
        def start_ring(i, n):
            pltpu.async_copy(dstr.at[pl.ds(eoff + i, 1)], idxD[n], semF)
            pltpu.async_copy(srcr.at[pl.ds(eoff + i, 1)], idxS[n], semF)
            pltpu.async_copy(dxr.at[pl.ds((eoff + i) * C, C)], dxc[n], semF)
            pltpu.async_copy(dzr.at[pl.ds((eoff + i) * C, C)], dzc[n], semF)

        def wait_ring(i, n):
            pltpu.make_async_copy(dstr.at[pl.ds(eoff + i, 1)], idxD[n],
                                  semF).wait()
            pltpu.make_async_copy(srcr.at[pl.ds(eoff + i, 1)], idxS[n],
                                  semF).wait()
            pltpu.make_async_copy(dxr.at[pl.ds((eoff + i) * C, C)], dxc[n],
                                  semF).wait()
            pltpu.make_async_copy(dzr.at[pl.ds((eoff + i) * C, C)], dzc[n],
                                  semF).wait()

        def start_gathers(g, n):
            pltpu.async_copy(ub_hbm.at[idxD[n].at[0]], ud[g], semG[g])
            pltpu.async_copy(ub_hbm.at[idxS[n].at[0]], us[g], semG[g])

        def wait_gathers(g, n):
            pltpu.make_async_copy(ub_hbm.at[idxD[n].at[0]], ud[g],
                                  semG[g]).wait()
            pltpu.make_async_copy(ub_hbm.at[idxS[n].at[0]], us[g],
                                  semG[g]).wait()

        def drain_dif(g, n):
            pltpu.make_async_copy(dif[g], acc.at[idxD[n].at[0]],
                                  semD[g]).wait()
            pltpu.make_async_copy(slpc[g], sacc.at[idxD[n].at[0]],
                                  semS[g]).wait()

        def drain_ndif(n):
            pltpu.make_async_copy(ndif, acc.at[idxS[n].at[0]], semN).wait()

        def compute(g, n):
            udg, usg, difg = ud[g], us[g], dif[g]
            dxn, dzn, slg = dxc[n], dzc[n], slpc[g]

            def ebody(e16, ecarry):
                ebase = e16 * 16
                sl = pl.ds(ebase, 16)
                wv = 1.0 / jnp.maximum(dxn[sl], 1e-6)
                slg[sl] = dzn[sl] * wv
                for k in range(16):
                    e = ebase + k
                    w = jnp.full((16,), wv[k], _f32)
                    for h in range(HF // 32):
                        a_d, b_d = plsc.unpack(
                            udg[e, pl.ds(h * 32, 32)],
                            format=plsc.PackFormat.INTERLEAVED,
                            preferred_element_type=_f32)
                        a_s, b_s = plsc.unpack(
                            usg[e, pl.ds(h * 32, 32)],
                            format=plsc.PackFormat.INTERLEAVED,
                            preferred_element_type=_f32)
                        t0 = (a_d - a_s) * w
                        t1 = (b_d - b_s) * w
                        difg[e, pl.ds(h * 32, 16)] = t0
                        ndif[e, pl.ds(h * 32, 16)] = -t0
                        difg[e, pl.ds(h * 32 + 16, 16)] = t1
                        ndif[e, pl.ds(h * 32 + 16, 16)] = -t1
                return ecarry
            lax.fori_loop(0, C // 16, ebody, 0)

        def half(k, j):
            i = NIB * k + j
            g, n = j % 2, j
            wait_gathers(g, n)

            if j >= 2:
                drain_dif(g, n - 2)
            else:
                @pl.when(k >= 1)
                def _():
                    drain_dif(g, (n - 2) % NIB)
            if j >= 1:
                drain_ndif(n - 1)
            else:
                @pl.when(k >= 1)
                def _():
                    drain_ndif(NIB - 1)

            compute(g, n)

            pltpu.async_copy(dif[g], acc.at[idxD[n].at[0]], semD[g], add=True)
            pltpu.async_copy(ndif, acc.at[idxS[n].at[0]], semN, add=True)
            pltpu.async_copy(slpc[g], sacc.at[idxD[n].at[0]], semS[g],
                             add=True)

            @pl.when(i + 2 < CH)
            def _():
                wait_ring(i + 2, (n + 2) % NIB)
                start_gathers(g, (n + 2) % NIB)

            @pl.when(i + 4 < CH)
            def _():
                start_ring(i + 4, (n + 4) % NIB)

        for n in range(4):
            start_ring(n, n)
        for n in range(2):
            wait_ring(n, n)
            start_gathers(n % 2, n)

        def body(k, carry):
            for j in range(NIB):
                half(k, j)
            return carry
        lax.fori_loop(0, CH // NIB, body, 0)

        drain_dif(0, NIB - 2)
        drain_dif(1, NIB - 1)
        drain_ndif(NIB - 1)

    def _combine(u_hbm, out_hbm):
        for b in range(RB):
            off = base + b * C
            pltpu.sync_copy(u_hbm.at[pl.ds(off, C)], difA)
            pltpu.sync_copy(acc.at[pl.ds(off, C)], difB)
            pltpu.sync_copy(sacc.at[pl.ds(off, C)], sbuf)

            def rbody(r16, carry):
                rbase = r16 * 16
                sv16 = sbuf[pl.ds(rbase, 16)] * gdt
                for k in range(16):
                    r = rbase + k
                    sv = jnp.full((16,), sv16[k], _f32)
                    for v in range(HF // 16):
                        col = pl.ds(v * 16, 16)
                        uu = difA[r, col]
                        difA[r, col] = uu - dtc * (uu * difB[r, col]) - sv
                return carry
            lax.fori_loop(0, C // 16, rbody, 0)
            pltpu.sync_copy(difA, out_hbm.at[pl.ds(off, C)])

    def _core(ub_hbm, u_hbm, out_hbm):
        def _zrow(r, carry):
            for v in range(HF // 16):
                difA[r, pl.ds(v * 16, 16)] = jnp.zeros((16,), _f32)
            return carry
        lax.fori_loop(0, C, _zrow, 0)

        def _zs(v, carry):
            sbuf[pl.ds(v * 16, 16)] = jnp.zeros((16,), _f32)
            return carry
        lax.fori_loop(0, C // 16, _zs, 0)

        for b in range(RB):
            off = base + b * C
            pltpu.sync_copy(difA, acc.at[pl.ds(off, C)])
            pltpu.sync_copy(sbuf, sacc.at[pl.ds(off, C)])

        plsc.subcore_barrier()
        _edge_phase(ub_hbm)
        plsc.subcore_barrier()
        _combine(u_hbm, out_hbm)

    @pl.when(c == 0)
    def _():
        _core(ub0, uh0, oh0)

    @pl.when(c == 1)
    def _():
        _core(ub1, uh1, oh1)


def kernel(u, edge_index, edge_attr, dt, g):
    src = edge_index[0].astype(_i32)
    dst = edge_index[1].astype(_i32)
    pad = E_PAD - E_EDGES
    dstr = jnp.pad(dst, (0, pad)).reshape(E_PAD // C, C)
    srcr = jnp.pad(src, (0, pad)).reshape(E_PAD // C, C)
    dxr = jnp.pad(edge_attr[:, 0], (0, pad), constant_values=1.0)
    dzr = jnp.pad(edge_attr[:, 1], (0, pad))
    u_p = jnp.pad(u, ((0, N_PAD - N_NODES), (0, 0)))
    uh = [u_p[:, :HF], u_p[:, HF:]]
    perm32 = jnp.arange(32).reshape(2, 16).T.reshape(-1)
    perm = jnp.concatenate([perm32, perm32 + 32])
    ub = [h[:, perm].astype(_bf16) for h in uh]
    dt16 = jnp.full((16,), dt, _f32)
    g16 = jnp.full((16,), g, _f32)
    oh0, oh1 = _gstep_sc(ub[0], ub[1], uh[0], uh[1], dstr, srcr, dxr, dzr,
                         dt16, g16)
    return jnp.concatenate([oh0[:N_NODES], oh1[:N_NODES]], axis=1)


# speedup vs baseline: 1.0612x; 1.0612x over previous
import functools

import jax
import jax.numpy as jnp
from jax import lax
from jax.experimental import pallas as pl
from jax.experimental.pallas import tpu as pltpu
from jax.experimental.pallas import tpu_sc as plsc

N_NODES = 10000
N_PAD = 10240
D_FEAT = 128
HF = 64
E_EDGES = 320000
E_PAD = 331776
C = 128
CH = E_PAD // (16 * C)
NIB = 6
RB = 5
DT_MIN = 0.01
DT_MAX = 2.0

_f32 = jnp.float32
_i32 = jnp.int32
_bf16 = jnp.bfloat16

_RING = []
for _ in range(NIB):
    _RING += [
        pltpu.VMEM((1, C), _i32),
        pltpu.VMEM((1, C), _i32),
        pltpu.VMEM((C,), _f32),
        pltpu.VMEM((C,), _f32),
    ]


@functools.partial(
    pl.kernel,
    out_type=[jax.ShapeDtypeStruct((N_PAD, HF), _f32) for _ in range(2)],
    mesh=plsc.VectorSubcoreMesh(core_axis_name="c", subcore_axis_name="s"),
    compiler_params=pltpu.CompilerParams(use_tc_tiling_on_sc=False,
                                         needs_layout_passes=False),
    scratch_types=[
        pltpu.VMEM((C, HF), _bf16),
        pltpu.VMEM((C, HF), _bf16),
        pltpu.VMEM((C, HF), _bf16),
        pltpu.VMEM((C, HF), _bf16),
        pltpu.VMEM((C, HF), _f32),
        pltpu.VMEM((C, HF), _f32),
        pltpu.VMEM((C, HF), _f32),
        pltpu.VMEM((C, HF), _f32),
        pltpu.VMEM((C,), _f32),
        pltpu.VMEM((C,), _f32),
        pltpu.VMEM((16,), _f32),
        pltpu.VMEM((16,), _f32),
        pltpu.VMEM((C,), _f32),
        pltpu.VMEM_SHARED((N_PAD, HF), _f32),
        pltpu.VMEM_SHARED((N_PAD,), _f32),
        pltpu.SemaphoreType.DMA,
        pltpu.SemaphoreType.DMA,
        pltpu.SemaphoreType.DMA,
        pltpu.SemaphoreType.DMA,
        pltpu.SemaphoreType.DMA,
        pltpu.SemaphoreType.DMA,
        pltpu.SemaphoreType.DMA,
        pltpu.SemaphoreType.DMA,
    ] + _RING,
)
def _gstep_sc(ub0, ub1, uh0, uh1, dstr, srcr, dxr, dzr, dt_h, g_h,
              oh0, oh1,
              udA, usA, udB, usB, difA, difB, ndifA, ndifB, slpcA, slpcB,
              dtb, gb, sbuf, acc, sacc,
              semF, semGA, semGB, semDA, semDB, semN, semSA, semSB, *ring):
    idxD = [ring[4 * n + 0] for n in range(NIB)]
    idxS = [ring[4 * n + 1] for n in range(NIB)]
    dxc = [ring[4 * n + 2] for n in range(NIB)]
    dzc = [ring[4 * n + 3] for n in range(NIB)]
    ud = [udA, udB]
    us = [usA, usB]
    dif = [difA, difB]
    ndif = [ndifA, ndifB]
    slpc = [slpcA, slpcB]
    semG = [semGA, semGB]
    semD = [semDA, semDB]
    semS = [semSA, semSB]

    c = lax.axis_index("c")
    s = lax.axis_index("s")

    pltpu.sync_copy(dt_h, dtb)
    pltpu.sync_copy(g_h, gb)
    dtc = jnp.minimum(jnp.maximum(dtb[...], DT_MIN), DT_MAX)
    gdt = gb[...] * dtc

    base = s * (RB * C)
    eoff = s * CH

    def _edge_phase(ub_hbm):
        def start_ring(i, n):
            pltpu.async_copy(dstr.at[pl.ds(eoff + i, 1)], idxD[n], semF)
            pltpu.async_copy(srcr.at[pl.ds(eoff + i, 1)], idxS[n], semF)
            pltpu.async_copy(dxr.at[pl.ds((eoff + i) * C, C)], dxc[n], semF)
            pltpu.async_copy(dzr.at[pl.ds((eoff + i) * C, C)], dzc[n], semF)

        def wait_ring(i, n):
            pltpu.make_async_copy(dstr.at[pl.ds(eoff + i, 1)], idxD[n],
                                  semF).wait()
            pltpu.make_async_copy(srcr.at[pl.ds(eoff + i, 1)], idxS[n],
                                  semF).wait()
            pltpu.make_async_copy(dxr.at[pl.ds((eoff + i) * C, C)], dxc[n],
                                  semF).wait()
            pltpu.make_async_copy(dzr.at[pl.ds((eoff + i) * C, C)], dzc[n],
                                  semF).wait()

        def start_gathers(g, n):
            pltpu.async_copy(ub_hbm.at[idxD[n].at[0]], ud[g], semG[g])
            pltpu.async_copy(ub_hbm.at[idxS[n].at[0]], us[g], semG[g])

        def wait_gathers(g, n):
            pltpu.make_async_copy(ub_hbm.at[idxD[n].at[0]], ud[g],
                                  semG[g]).wait()
            pltpu.make_async_copy(ub_hbm.at[idxS[n].at[0]], us[g],
                                  semG[g]).wait()

        def drain_dif(g, n):
            pltpu.make_async_copy(dif[g], acc.at[idxD[n].at[0]],
                                  semD[g]).wait()
            pltpu.make_async_copy(ndif[g], acc.at[idxS[n].at[0]],
                                  semD[g]).wait()
            pltpu.make_async_copy(slpc[g], sacc.at[idxD[n].at[0]],
                                  semS[g]).wait()

        def compute(g, n):
            udg, usg, difg, ndifg = ud[g], us[g], dif[g], ndif[g]
            dxn, dzn, slg = dxc[n], dzc[n], slpc[g]

            def ebody(e16, ecarry):
                ebase = e16 * 16
                sl = pl.ds(ebase, 16)
                wv = 1.0 / jnp.maximum(dxn[sl], 1e-6)
                slg[sl] = dzn[sl] * wv
                for k in range(16):
                    e = ebase + k
                    w = jnp.full((16,), wv[k], _f32)
                    for h in range(HF // 32):
                        a_d, b_d = plsc.unpack(
                            udg[e, pl.ds(h * 32, 32)],
                            format=plsc.PackFormat.INTERLEAVED,
                            preferred_element_type=_f32)
                        a_s, b_s = plsc.unpack(
                            usg[e, pl.ds(h * 32, 32)],
                            format=plsc.PackFormat.INTERLEAVED,
                            preferred_element_type=_f32)
                        t0 = (a_d - a_s) * w
                        t1 = (b_d - b_s) * w
                        difg[e, pl.ds(h * 32, 16)] = t0
                        ndifg[e, pl.ds(h * 32, 16)] = -t0
                        difg[e, pl.ds(h * 32 + 16, 16)] = t1
                        ndifg[e, pl.ds(h * 32 + 16, 16)] = -t1
                return ecarry
            lax.fori_loop(0, C // 16, ebody, 0)

        def half(k, j):
            i = NIB * k + j
            g, n = j % 2, j
            wait_gathers(g, n)

            if j >= 2:
                drain_dif(g, n - 2)
            else:
                @pl.when(k >= 1)
                def _():
                    drain_dif(g, (n - 2) % NIB)

            compute(g, n)

            pltpu.async_copy(dif[g], acc.at[idxD[n].at[0]], semD[g], add=True)
            pltpu.async_copy(ndif[g], acc.at[idxS[n].at[0]], semD[g],
                             add=True)
            pltpu.async_copy(slpc[g], sacc.at[idxD[n].at[0]], semS[g],
                             add=True)

            @pl.when(i + 2 < CH)
            def _():
                wait_ring(i + 2, (n + 2) % NIB)
                start_gathers(g, (n + 2) % NIB)

            @pl.when(i + 4 < CH)
            def _():
                start_ring(i + 4, (n + 4) % NIB)

        for n in range(4):
            start_ring(n, n)
        for n in range(2):
            wait_ring(n, n)
            start_gathers(n % 2, n)

        def body(k, carry):
            for j in range(NIB):
                half(k, j)
            return carry
        lax.fori_loop(0, CH // NIB, body, 0)

        drain_dif(0, NIB - 2)
        drain_dif(1, NIB - 1)

    def _combine(u_hbm, out_hbm):
        for b in range(RB):
            off = base + b * C
            pltpu.sync_copy(u_hbm.at[pl.ds(off, C)], difA)
            pltpu.sync_copy(acc.at[pl.ds(off, C)], difB)
            pltpu.sync_copy(sacc.at[pl.ds(off, C)], sbuf)

            def rbody(r16, carry):
                rbase = r16 * 16
                sv16 = sbuf[pl.ds(rbase, 16)] * gdt
                for k in range(16):
                    r = rbase + k
                    sv = jnp.full((16,), sv16[k], _f32)
                    for v in range(HF // 16):
                        col = pl.ds(v * 16, 16)
                        uu = difA[r, col]
                        difA[r, col] = uu - dtc * (uu * difB[r, col]) - sv
                return carry
            lax.fori_loop(0, C // 16, rbody, 0)
            pltpu.sync_copy(difA, out_hbm.at[pl.ds(off, C)])

    def _core(ub_hbm, u_hbm, out_hbm):
        def _zrow(r, carry):
            for v in range(HF // 16):
                difA[r, pl.ds(v * 16, 16)] = jnp.zeros((16,), _f32)
            return carry
        lax.fori_loop(0, C, _zrow, 0)

        def _zs(v, carry):
            sbuf[pl.ds(v * 16, 16)] = jnp.zeros((16,), _f32)
            return carry
        lax.fori_loop(0, C // 16, _zs, 0)

        for b in range(RB):
            off = base + b * C
            pltpu.sync_copy(difA, acc.at[pl.ds(off, C)])
            pltpu.sync_copy(sbuf, sacc.at[pl.ds(off, C)])

        plsc.subcore_barrier()
        _edge_phase(ub_hbm)
        plsc.subcore_barrier()
        _combine(u_hbm, out_hbm)

    @pl.when(c == 0)
    def _():
        _core(ub0, uh0, oh0)

    @pl.when(c == 1)
    def _():
        _core(ub1, uh1, oh1)


def kernel(u, edge_index, edge_attr, dt, g):
    src = edge_index[0].astype(_i32)
    dst = edge_index[1].astype(_i32)
    pad = E_PAD - E_EDGES
    dstr = jnp.pad(dst, (0, pad)).reshape(E_PAD // C, C)
    srcr = jnp.pad(src, (0, pad)).reshape(E_PAD // C, C)
    dxr = jnp.pad(edge_attr[:, 0], (0, pad), constant_values=1.0)
    dzr = jnp.pad(edge_attr[:, 1], (0, pad))
    u_p = jnp.pad(u, ((0, N_PAD - N_NODES), (0, 0)))
    uh = [u_p[:, :HF], u_p[:, HF:]]
    perm32 = jnp.arange(32).reshape(2, 16).T.reshape(-1)
    perm = jnp.concatenate([perm32, perm32 + 32])
    ub = [h[:, perm].astype(_bf16) for h in uh]
    dt16 = jnp.full((16,), dt, _f32)
    g16 = jnp.full((16,), g, _f32)
    oh0, oh1 = _gstep_sc(ub[0], ub[1], uh[0], uh[1], dstr, srcr, dxr, dzr,
                         dt16, g16)
    return jnp.concatenate([oh0[:N_NODES], oh1[:N_NODES]], axis=1)


# speedup vs baseline: 1.0647x; 1.0034x over previous
import functools

import jax
import jax.numpy as jnp
from jax import lax
from jax.experimental import pallas as pl
from jax.experimental.pallas import tpu as pltpu
from jax.experimental.pallas import tpu_sc as plsc

N_NODES = 10000
N_PAD = 10240
D_FEAT = 128
HF = 64
E_EDGES = 320000
E_PAD = 331776
C = 128
CH = E_PAD // (16 * C)
NIB = 6
RB = 5
DT_MIN = 0.01
DT_MAX = 2.0

_f32 = jnp.float32
_i32 = jnp.int32
_bf16 = jnp.bfloat16

_RING = []
for _ in range(NIB):
    _RING += [
        pltpu.VMEM((1, C), _i32),
        pltpu.VMEM((1, C), _i32),
        pltpu.VMEM((C,), _f32),
        pltpu.VMEM((C,), _f32),
    ]


@functools.partial(
    pl.kernel,
    out_type=[jax.ShapeDtypeStruct((N_PAD, HF), _f32) for _ in range(2)],
    mesh=plsc.VectorSubcoreMesh(core_axis_name="c", subcore_axis_name="s"),
    compiler_params=pltpu.CompilerParams(use_tc_tiling_on_sc=False,
                                         needs_layout_passes=False),
    scratch_types=[
        pltpu.VMEM((C, HF), _bf16),
        pltpu.VMEM((C, HF), _bf16),
        pltpu.VMEM((C, HF), _bf16),
        pltpu.VMEM((C, HF), _bf16),
        pltpu.VMEM((C, HF), _bf16),
        pltpu.VMEM((C, HF), _bf16),
        pltpu.VMEM((C, HF), _f32),
        pltpu.VMEM((C, HF), _f32),
        pltpu.VMEM((C, HF), _f32),
        pltpu.VMEM((C, HF), _f32),
        pltpu.VMEM((C,), _f32),
        pltpu.VMEM((C,), _f32),
        pltpu.VMEM((16,), _f32),
        pltpu.VMEM((16,), _f32),
        pltpu.VMEM((C,), _f32),
        pltpu.VMEM_SHARED((N_PAD, HF), _f32),
        pltpu.VMEM_SHARED((N_PAD,), _f32),
        pltpu.SemaphoreType.DMA,
        pltpu.SemaphoreType.DMA,
        pltpu.SemaphoreType.DMA,
        pltpu.SemaphoreType.DMA,
        pltpu.SemaphoreType.DMA,
        pltpu.SemaphoreType.DMA,
        pltpu.SemaphoreType.DMA,
        pltpu.SemaphoreType.DMA,
    ] + _RING,
)
def _gstep_sc(ub0, ub1, uh0, uh1, dstr, srcr, dxr, dzr, dt_h, g_h,
              oh0, oh1,
              udA, usA, udB, usB, udC, usC, difA, difB, ndifA, ndifB,
              slpcA, slpcB,
              dtb, gb, sbuf, acc, sacc,
              semF, semGA, semGB, semDA, semDB, semN, semSA, semSB, *ring):
    idxD = [ring[4 * n + 0] for n in range(NIB)]
    idxS = [ring[4 * n + 1] for n in range(NIB)]
    dxc = [ring[4 * n + 2] for n in range(NIB)]
    dzc = [ring[4 * n + 3] for n in range(NIB)]
    ud = [udA, udB, udC]
    us = [usA, usB, usC]
    dif = [difA, difB]
    ndif = [ndifA, ndifB]
    slpc = [slpcA, slpcB]
    semG = [semGA, semGB, semN]
    semD = [semDA, semDB]
    semS = [semSA, semSB]

    c = lax.axis_index("c")
    s = lax.axis_index("s")

    pltpu.sync_copy(dt_h, dtb)
    pltpu.sync_copy(g_h, gb)
    dtc = jnp.minimum(jnp.maximum(dtb[...], DT_MIN), DT_MAX)
    gdt = gb[...] * dtc

    base = s * (RB * C)
    eoff = s * CH

    def _edge_phase(ub_hbm):
        def start_ring(i, n):
            pltpu.async_copy(dstr.at[pl.ds(eoff + i, 1)], idxD[n], semF)
            pltpu.async_copy(srcr.at[pl.ds(eoff + i, 1)], idxS[n], semF)
            pltpu.async_copy(dxr.at[pl.ds((eoff + i) * C, C)], dxc[n], semF)
            pltpu.async_copy(dzr.at[pl.ds((eoff + i) * C, C)], dzc[n], semF)

        def wait_ring(i, n):
            pltpu.make_async_copy(dstr.at[pl.ds(eoff + i, 1)], idxD[n],
                                  semF).wait()
            pltpu.make_async_copy(srcr.at[pl.ds(eoff + i, 1)], idxS[n],
                                  semF).wait()
            pltpu.make_async_copy(dxr.at[pl.ds((eoff + i) * C, C)], dxc[n],
                                  semF).wait()
            pltpu.make_async_copy(dzr.at[pl.ds((eoff + i) * C, C)], dzc[n],
                                  semF).wait()

        def start_gathers(g, n):
            pltpu.async_copy(ub_hbm.at[idxD[n].at[0]], ud[g], semG[g])
            pltpu.async_copy(ub_hbm.at[idxS[n].at[0]], us[g], semG[g])

        def wait_gathers(g, n):
            pltpu.make_async_copy(ub_hbm.at[idxD[n].at[0]], ud[g],
                                  semG[g]).wait()
            pltpu.make_async_copy(ub_hbm.at[idxS[n].at[0]], us[g],
                                  semG[g]).wait()

        def drain_dif(g, n):
            pltpu.make_async_copy(dif[g], acc.at[idxD[n].at[0]],
                                  semD[g]).wait()
            pltpu.make_async_copy(ndif[g], acc.at[idxS[n].at[0]],
                                  semD[g]).wait()
            pltpu.make_async_copy(slpc[g], sacc.at[idxD[n].at[0]],
                                  semS[g]).wait()

        def compute(g, gg, n):
            udg, usg, difg, ndifg = ud[gg], us[gg], dif[g], ndif[g]
            dxn, dzn, slg = dxc[n], dzc[n], slpc[g]

            def ebody(e16, ecarry):
                ebase = e16 * 16
                sl = pl.ds(ebase, 16)
                wv = 1.0 / jnp.maximum(dxn[sl], 1e-6)
                slg[sl] = dzn[sl] * wv
                for k in range(16):
                    e = ebase + k
                    w = jnp.full((16,), wv[k], _f32)
                    for h in range(HF // 32):
                        a_d, b_d = plsc.unpack(
                            udg[e, pl.ds(h * 32, 32)],
                            format=plsc.PackFormat.INTERLEAVED,
                            preferred_element_type=_f32)
                        a_s, b_s = plsc.unpack(
                            usg[e, pl.ds(h * 32, 32)],
                            format=plsc.PackFormat.INTERLEAVED,
                            preferred_element_type=_f32)
                        t0 = (a_d - a_s) * w
                        t1 = (b_d - b_s) * w
                        difg[e, pl.ds(h * 32, 16)] = t0
                        ndifg[e, pl.ds(h * 32, 16)] = -t0
                        difg[e, pl.ds(h * 32 + 16, 16)] = t1
                        ndifg[e, pl.ds(h * 32 + 16, 16)] = -t1
                return ecarry
            lax.fori_loop(0, C // 16, ebody, 0)

        def half(k, j):
            i = NIB * k + j
            g, gg, n = j % 2, j % 3, j
            wait_gathers(gg, n)

            if j >= 2:
                drain_dif(g, n - 2)
            else:
                @pl.when(k >= 1)
                def _():
                    drain_dif(g, (n - 2) % NIB)

            compute(g, gg, n)

            pltpu.async_copy(dif[g], acc.at[idxD[n].at[0]], semD[g], add=True)
            pltpu.async_copy(ndif[g], acc.at[idxS[n].at[0]], semD[g],
                             add=True)
            pltpu.async_copy(slpc[g], sacc.at[idxD[n].at[0]], semS[g],
                             add=True)

            @pl.when(i + 3 < CH)
            def _():
                wait_ring(i + 3, (n + 3) % NIB)
                start_gathers(gg, (n + 3) % NIB)

            @pl.when(i + 4 < CH)
            def _():
                start_ring(i + 4, (n + 4) % NIB)

        for n in range(4):
            start_ring(n, n)
        for n in range(3):
            wait_ring(n, n)
            start_gathers(n % 3, n)

        def body(k, carry):
            for j in range(NIB):
                half(k, j)
            return carry
        lax.fori_loop(0, CH // NIB, body, 0)

        drain_dif(0, NIB - 2)
        drain_dif(1, NIB - 1)

    def _combine(u_hbm, out_hbm):
        for b in range(RB):
            off = base + b * C
            pltpu.sync_copy(u_hbm.at[pl.ds(off, C)], difA)
            pltpu.sync_copy(acc.at[pl.ds(off, C)], difB)
            pltpu.sync_copy(sacc.at[pl.ds(off, C)], sbuf)

            def rbody(r16, carry):
                rbase = r16 * 16
                sv16 = sbuf[pl.ds(rbase, 16)] * gdt
                for k in range(16):
                    r = rbase + k
                    sv = jnp.full((16,), sv16[k], _f32)
                    for v in range(HF // 16):
                        col = pl.ds(v * 16, 16)
                        uu = difA[r, col]
                        difA[r, col] = uu - dtc * (uu * difB[r, col]) - sv
                return carry
            lax.fori_loop(0, C // 16, rbody, 0)
            pltpu.sync_copy(difA, out_hbm.at[pl.ds(off, C)])

    def _core(ub_hbm, u_hbm, out_hbm):
        def _zrow(r, carry):
            for v in range(HF // 16):
                difA[r, pl.ds(v * 16, 16)] = jnp.zeros((16,), _f32)
            return carry
        lax.fori_loop(0, C, _zrow, 0)

        def _zs(v, carry):
            sbuf[pl.ds(v * 16, 16)] = jnp.zeros((16,), _f32)
            return carry
        lax.fori_loop(0, C // 16, _zs, 0)

        for b in range(RB):
            off = base + b * C
            pltpu.sync_copy(difA, acc.at[pl.ds(off, C)])
            pltpu.sync_copy(sbuf, sacc.at[pl.ds(off, C)])

        plsc.subcore_barrier()
        _edge_phase(ub_hbm)
        plsc.subcore_barrier()
        _combine(u_hbm, out_hbm)

    @pl.when(c == 0)
    def _():
        _core(ub0, uh0, oh0)

    @pl.when(c == 1)
    def _():
        _core(ub1, uh1, oh1)


def kernel(u, edge_index, edge_attr, dt, g):
    src = edge_index[0].astype(_i32)
    dst = edge_index[1].astype(_i32)
    pad = E_PAD - E_EDGES
    dstr = jnp.pad(dst, (0, pad)).reshape(E_PAD // C, C)
    srcr = jnp.pad(src, (0, pad)).reshape(E_PAD // C, C)
    dxr = jnp.pad(edge_attr[:, 0], (0, pad), constant_values=1.0)
    dzr = jnp.pad(edge_attr[:, 1], (0, pad))
    u_p = jnp.pad(u, ((0, N_PAD - N_NODES), (0, 0)))
    uh = [u_p[:, :HF], u_p[:, HF:]]
    perm32 = jnp.arange(32).reshape(2, 16).T.reshape(-1)
    perm = jnp.concatenate([perm32, perm32 + 32])
    ub = [h[:, perm].astype(_bf16) for h in uh]
    dt16 = jnp.full((16,), dt, _f32)
    g16 = jnp.full((16,), g, _f32)
    oh0, oh1 = _gstep_sc(ub[0], ub[1], uh[0], uh[1], dstr, srcr, dxr, dzr,
                         dt16, g16)
    return jnp.concatenate([oh0[:N_NODES], oh1[:N_NODES]], axis=1)


# speedup vs baseline: 1.0787x; 1.0131x over previous
import functools

import jax
import jax.numpy as jnp
from jax import lax
from jax.experimental import pallas as pl
from jax.experimental.pallas import tpu as pltpu
from jax.experimental.pallas import tpu_sc as plsc

N_NODES = 10000
N_PAD = 10240
D_FEAT = 128
HQ = 32
E_EDGES = 320000
E_PAD = 327680
C = 128
CH = E_PAD // (16 * C)
RB = 5
DT_MIN = 0.01
DT_MAX = 2.0

_f32 = jnp.float32
_i32 = jnp.int32
_bf16 = jnp.bfloat16


@functools.partial(
    pl.kernel,
    out_type=[jax.ShapeDtypeStruct((N_PAD, HQ), _f32) for _ in range(4)],
    mesh=plsc.VectorSubcoreMesh(core_axis_name="c", subcore_axis_name="s"),
    compiler_params=pltpu.CompilerParams(use_tc_tiling_on_sc=False,
                                         needs_layout_passes=False),
    scratch_types=[
        pltpu.VMEM((CH, C), _i32),
        pltpu.VMEM((CH, C), _i32),
        pltpu.VMEM((C, HQ), _bf16),
        pltpu.VMEM((C, HQ), _bf16),
        pltpu.VMEM((C, HQ), _bf16),
        pltpu.VMEM((C, HQ), _bf16),
        pltpu.VMEM((C, HQ), _f32),
        pltpu.VMEM((C, HQ), _f32),
        pltpu.VMEM((C, HQ), _f32),
        pltpu.VMEM((C, HQ), _f32),
        pltpu.VMEM((C, HQ), _f32),
        pltpu.VMEM((C, HQ), _f32),
        pltpu.VMEM((C,), _f32),
        pltpu.VMEM((C,), _f32),
        pltpu.VMEM((C,), _f32),
        pltpu.VMEM((C,), _f32),
        pltpu.VMEM((C,), _f32),
        pltpu.VMEM((C,), _f32),
        pltpu.VMEM((16,), _f32),
        pltpu.VMEM((16,), _f32),
        pltpu.VMEM((C,), _f32),
        pltpu.VMEM_SHARED((N_PAD, HQ), _f32),
        pltpu.VMEM_SHARED((N_PAD,), _f32),
        pltpu.SemaphoreType.DMA,
        pltpu.SemaphoreType.DMA,
        pltpu.SemaphoreType.DMA,
        pltpu.SemaphoreType.DMA,
    ],
)
def _gstep_sc(uq0, uq1, uq2, uq3, ub0, ub1, ub2, ub3, dstr, srcr, dxr, dzr,
              dt_h, g_h,
              oq0, oq1, oq2, oq3,
              dsti, srci, udA, usA, udB, usB, ucA, ucB,
              difA, ndifA, difB, ndifB,
              dxcA, dxcB, dzcA, dzcB, slpcA, slpcB, dtb, gb, sbuf,
              acc, sacc, semGA, semGB, semSA, semSB):
    c = lax.axis_index("c")
    s = lax.axis_index("s")

    pltpu.sync_copy(dt_h, dtb)
    pltpu.sync_copy(g_h, gb)
    dtc = jnp.minimum(jnp.maximum(dtb[...], DT_MIN), DT_MAX)
    gdt = gb[...] * dtc

    base = s * (RB * C)

    eoff = s * CH
    pltpu.sync_copy(dstr.at[pl.ds(eoff, CH)], dsti)
    pltpu.sync_copy(srcr.at[pl.ds(eoff, CH)], srci)

    def _edge_phase(ub_hbm, do_s):
        def start_fetch(i, ud, us, dxc, dzc, semG):
            pltpu.async_copy(ub_hbm.at[dsti.at[i]], ud, semG)
            pltpu.async_copy(ub_hbm.at[srci.at[i]], us, semG)
            pltpu.async_copy(dxr.at[pl.ds((eoff + i) * C, C)], dxc, semG)
            if do_s:
                pltpu.async_copy(dzr.at[pl.ds((eoff + i) * C, C)], dzc, semG)

        def wait_fetch(i, ud, us, dxc, dzc, semG):
            pltpu.make_async_copy(ub_hbm.at[dsti.at[i]], ud, semG).wait()
            pltpu.make_async_copy(ub_hbm.at[srci.at[i]], us, semG).wait()
            pltpu.make_async_copy(dxr.at[pl.ds((eoff + i) * C, C)], dxc,
                                  semG).wait()
            if do_s:
                pltpu.make_async_copy(dzr.at[pl.ds((eoff + i) * C, C)], dzc,
                                      semG).wait()

        def drain_scatters(i, dif, ndif, slpc, semS):
            pltpu.make_async_copy(dif, acc.at[dsti.at[i]], semS).wait()
            pltpu.make_async_copy(ndif, acc.at[srci.at[i]], semS).wait()
            if do_s:
                pltpu.make_async_copy(slpc, sacc.at[dsti.at[i]], semS).wait()

        def compute(ud, us, dxc, dzc, slpc, dif, ndif):
            def ebody(e16, ecarry):
                ebase = e16 * 16
                sl = pl.ds(ebase, 16)
                wv = 1.0 / jnp.maximum(dxc[sl], 1e-6)
                if do_s:
                    slpc[sl] = dzc[sl] * wv
                for k in range(16):
                    e = ebase + k
                    w = jnp.full((16,), wv[k], _f32)
                    ad, bd = plsc.unpack(
                        ud[e, :], format=plsc.PackFormat.INTERLEAVED,
                        preferred_element_type=_f32)
                    asr, bs = plsc.unpack(
                        us[e, :], format=plsc.PackFormat.INTERLEAVED,
                        preferred_element_type=_f32)
                    t0 = (ad - asr) * w
                    t1 = (bd - bs) * w
                    dif[e, pl.ds(0, 16)] = t0
                    ndif[e, pl.ds(0, 16)] = -t0
                    dif[e, pl.ds(16, 16)] = t1
                    ndif[e, pl.ds(16, 16)] = -t1
                return ecarry
            lax.fori_loop(0, C // 16, ebody, 0)

        def half(k, i, ud, us, dxc, dzc, slpc, dif, ndif, semG, semS):
            wait_fetch(i, ud, us, dxc, dzc, semG)

            @pl.when(k >= 1)
            def _():
                drain_scatters(i, dif, ndif, slpc, semS)

            compute(ud, us, dxc, dzc, slpc, dif, ndif)

            pltpu.async_copy(dif, acc.at[dsti.at[i]], semS, add=True)
            pltpu.async_copy(ndif, acc.at[srci.at[i]], semS, add=True)
            if do_s:
                pltpu.async_copy(slpc, sacc.at[dsti.at[i]], semS, add=True)

            @pl.when(i + 2 < CH)
            def _():
                start_fetch(i + 2, ud, us, dxc, dzc, semG)

        start_fetch(0, udA, usA, dxcA, dzcA, semGA)
        start_fetch(1, udB, usB, dxcB, dzcB, semGB)

        def body(k, carry):
            half(k, 2 * k, udA, usA, dxcA, dzcA, slpcA, difA, ndifA,
                 semGA, semSA)
            half(k, 2 * k + 1, udB, usB, dxcB, dzcB, slpcB, difB, ndifB,
                 semGB, semSB)
            return carry
        lax.fori_loop(0, CH // 2, body, 0)

        drain_scatters(CH - 2, difA, ndifA, slpcA, semSA)
        drain_scatters(CH - 1, difB, ndifB, slpcB, semSB)

    def _combine(u_hbm, out_hbm):
        for b in range(RB):
            off = base + b * C
            pltpu.sync_copy(u_hbm.at[pl.ds(off, C)], ucA)
            pltpu.sync_copy(acc.at[pl.ds(off, C)], ucB)
            pltpu.sync_copy(sacc.at[pl.ds(off, C)], sbuf)

            def rbody(r16, carry):
                rbase = r16 * 16
                sv16 = sbuf[pl.ds(rbase, 16)] * gdt
                for k in range(16):
                    r = rbase + k
                    sv = jnp.full((16,), sv16[k], _f32)
                    for v in range(HQ // 16):
                        col = pl.ds(v * 16, 16)
                        uu = ucA[r, col]
                        ucA[r, col] = uu - dtc * (uu * ucB[r, col]) - sv
                return carry
            lax.fori_loop(0, C // 16, rbody, 0)
            pltpu.sync_copy(ucA, out_hbm.at[pl.ds(off, C)])

    def _pass(ub_hbm, u_hbm, out_hbm, do_s):
        def _zrow(r, carry):
            for v in range(HQ // 16):
                difA[r, pl.ds(v * 16, 16)] = jnp.zeros((16,), _f32)
            return carry
        lax.fori_loop(0, C, _zrow, 0)

        for b in range(RB):
            off = base + b * C
            pltpu.sync_copy(difA, acc.at[pl.ds(off, C)])

        if do_s:
            def _zs(v, carry):
                sbuf[pl.ds(v * 16, 16)] = jnp.zeros((16,), _f32)
                return carry
            lax.fori_loop(0, C // 16, _zs, 0)
            for b in range(RB):
                off = base + b * C
                pltpu.sync_copy(sbuf, sacc.at[pl.ds(off, C)])

        plsc.subcore_barrier()
        _edge_phase(ub_hbm, do_s)
        plsc.subcore_barrier()
        _combine(u_hbm, out_hbm)

    @pl.when(c == 0)
    def _():
        _pass(ub0, uq0, oq0, True)
        _pass(ub1, uq1, oq1, False)

    @pl.when(c == 1)
    def _():
        _pass(ub2, uq2, oq2, True)
        _pass(ub3, uq3, oq3, False)


def kernel(u, edge_index, edge_attr, dt, g):
    src = edge_index[0].astype(_i32)
    dst = edge_index[1].astype(_i32)
    pad = E_PAD - E_EDGES
    dstr = jnp.pad(dst, (0, pad)).reshape(E_PAD // C, C)
    srcr = jnp.pad(src, (0, pad)).reshape(E_PAD // C, C)
    dxr = jnp.pad(edge_attr[:, 0], (0, pad), constant_values=1.0)
    dzr = jnp.pad(edge_attr[:, 1], (0, pad))
    u_p = jnp.pad(u, ((0, N_PAD - N_NODES), (0, 0)))
    uq = [u_p[:, i * HQ:(i + 1) * HQ] for i in range(4)]
    perm = jnp.arange(HQ).reshape(2, HQ // 2).T.reshape(-1)
    ub = [q[:, perm].astype(_bf16) for q in uq]
    dt16 = jnp.full((16,), dt, _f32)
    g16 = jnp.full((16,), g, _f32)
    oq = _gstep_sc(*uq, *ub, dstr, srcr, dxr, dzr, dt16, g16)
    return jnp.concatenate([o[:N_NODES] for o in oq], axis=1)
